# Initial kernel scaffold; baseline (speedup 1.0000x reference)
import jax
import jax.numpy as jnp
from jax.experimental import pallas as pl


def kernel(x, x_v, W, b):
    raise NotImplementedError("write your pallas kernel here")


# speedup vs baseline: 1.4733x; 1.4733x over previous
import functools

import jax
import jax.numpy as jnp
from jax import lax
from jax.experimental import pallas as pl
from jax.experimental.pallas import tpu as pltpu
from jax.experimental.pallas import tpu_sc as plsc

B, N, D, K, E = 4, 4096, 128, 27, 256
BQ = 512
BM = 512


def _dist_body(xq_ref, xk_ref, out_ref):
    q = xq_ref[0]
    k = xk_ref[0]
    qn = jnp.sum(q * q, axis=1, keepdims=True)
    kn = jnp.sum(k * k, axis=1, keepdims=True)
    dot = jax.lax.dot_general(q, k, (((1,), (1,)), ((), ())),
                              preferred_element_type=jnp.float32)
    out_ref[0] = -((qn + kn.T) - 2.0 * dot)


def _neg_dmat(x_v_pad):
    return pl.pallas_call(
        _dist_body,
        grid=(B, N // BQ),
        in_specs=[
            pl.BlockSpec((1, BQ, 8), lambda b, i: (b, i, 0)),
            pl.BlockSpec((1, N, 8), lambda b, i: (b, 0, 0)),
        ],
        out_specs=pl.BlockSpec((1, BQ, N), lambda b, i: (b, i, 0)),
        out_shape=jax.ShapeDtypeStruct((B, N, N), jnp.float32),
    )(x_v_pad, x_v_pad)


_info = plsc.get_sparse_core_info()
_NC, _NS = _info.num_cores, _info.num_subcores
_NW = _NC * _NS
_ROWS = B * N * K
_RPW = _ROWS // _NW
_CHUNK = 512


def _make_gather():
    mesh = plsc.VectorSubcoreMesh(core_axis_name="c", subcore_axis_name="s")

    @functools.partial(
        pl.kernel, mesh=mesh,
        out_type=jax.ShapeDtypeStruct((_ROWS, D), jnp.float32),
        scratch_types=[
            pltpu.VMEM((_CHUNK,), jnp.int32),
            pltpu.VMEM((_CHUNK, D), jnp.float32),
            pltpu.SemaphoreType.DMA,
        ],
    )
    def gather_k(table_hbm, idx_hbm, out_hbm, idx_v, rows_v, sem):
        wid = lax.axis_index("s") * _NC + lax.axis_index("c")
        base = wid * _RPW

        def body(i, _):
            off = base + i * _CHUNK
            pltpu.sync_copy(idx_hbm.at[pl.ds(off, _CHUNK)], idx_v)
            pltpu.async_copy(table_hbm.at[idx_v], rows_v, sem).wait()
            pltpu.sync_copy(rows_v, out_hbm.at[pl.ds(off, _CHUNK)])
            return ()

        lax.fori_loop(0, _RPW // _CHUNK, body, (), unroll=False)

    return gather_k


_gather = _make_gather()


def _mm_body(g_ref, w_ref, b_ref, out_ref):
    out_ref[...] = (
        jax.lax.dot_general(g_ref[...], w_ref[...], (((1,), (1,)), ((), ())),
                            preferred_element_type=jnp.float32)
        + b_ref[...]
    )


def _linear(gathered_flat, W, b):
    return pl.pallas_call(
        _mm_body,
        grid=(B * N // BM,),
        in_specs=[
            pl.BlockSpec((BM, K * D), lambda i: (i, 0)),
            pl.BlockSpec((E, K * D), lambda i: (0, 0)),
            pl.BlockSpec((1, E), lambda i: (0, 0)),
        ],
        out_specs=pl.BlockSpec((BM, E), lambda i: (i, 0)),
        out_shape=jax.ShapeDtypeStruct((B * N, E), jnp.float32),
    )(gathered_flat, W, b.reshape(1, E))


def kernel(x, x_v, W, b):
    x_v_pad = jnp.pad(x_v, ((0, 0), (0, 0), (0, 5)))
    neg_d = _neg_dmat(x_v_pad)
    _, idx = jax.lax.top_k(neg_d, K)
    idx_g = (idx + (jnp.arange(B, dtype=jnp.int32) * N)[:, None, None])
    table = x.reshape(B * N, D)
    gathered = _gather(table, idx_g.reshape(-1))
    out = _linear(gathered.reshape(B * N, K * D), W, b)
    return out.reshape(B, N, E)
